# baseline (speedup 1.0000x reference)
import dataclasses
import functools

import jax
import jax.numpy as jnp
from jax import lax
from jax.experimental import pallas as pl
from jax.experimental.pallas import tpu as pltpu
from jax.experimental.pallas import tpu_sc as plsc

B, D, P, A, M = 16384, 512, 256, 32, 4096
BETA_LONG, BETA_SHORT, ALPHA_LP = 0.995, 0.9, 0.5

ERR_TILE = 2048


def _err_body(obs_ref, nobs_ref, act_ref, we_ref, be_ref, wf1_ref, wf2_ref,
              bf_ref, err_ref):
    obs = obs_ref[...]
    nobs = nobs_ref[...]
    phi_t = jnp.maximum(obs @ we_ref[...] + be_ref[...], 0.0)
    phi_tp1 = jnp.maximum(nobs @ we_ref[...] + be_ref[...], 0.0)
    pred = phi_t @ wf1_ref[...] + act_ref[...] @ wf2_ref[...] + bf_ref[...]
    d = pred - phi_tp1
    err_ref[...] = jnp.sum(d * d, axis=1) * (1.0 / P)


def _err_tc(obs, next_obs, actions, W_enc, b_enc, W_fwd, b_fwd):
    grid = B // ERR_TILE
    return pl.pallas_call(
        _err_body,
        grid=(grid,),
        in_specs=[
            pl.BlockSpec((ERR_TILE, D), lambda i: (i, 0)),
            pl.BlockSpec((ERR_TILE, D), lambda i: (i, 0)),
            pl.BlockSpec((ERR_TILE, A), lambda i: (i, 0)),
            pl.BlockSpec((D, P), lambda i: (0, 0)),
            pl.BlockSpec((1, P), lambda i: (0, 0)),
            pl.BlockSpec((P, P), lambda i: (0, 0)),
            pl.BlockSpec((A, P), lambda i: (0, 0)),
            pl.BlockSpec((1, P), lambda i: (0, 0)),
        ],
        out_specs=pl.BlockSpec((ERR_TILE,), lambda i: (i,)),
        out_shape=jax.ShapeDtypeStruct((B,), jnp.float32),
    )(obs, next_obs, actions, W_enc, b_enc.reshape(1, P), W_fwd[:P],
      W_fwd[P:], b_fwd.reshape(1, P))


NS = 16
ROWS = B // 128
RPW = ROWS // NS
MS = M // NS


def _sc_middle(err2d, rids2d, ema_long, ema_short, counts, prev_ms):
    mesh = plsc.VectorSubcoreMesh(core_axis_name="c", subcore_axis_name="s")
    cp = pltpu.CompilerParams()
    if "needs_layout_passes" in pltpu.CompilerParams.__dataclass_fields__:
        cp = dataclasses.replace(cp, needs_layout_passes=False)

    @functools.partial(
        pl.kernel, mesh=mesh, compiler_params=cp,
        out_type=jax.ShapeDtypeStruct((ROWS, 128), jnp.float32),
        scratch_types=[
            pltpu.VMEM((RPW, 128), jnp.int32),
            pltpu.VMEM((RPW, 128), jnp.float32),
            pltpu.VMEM((128,), jnp.float32),
            pltpu.VMEM((MS,), jnp.float32),
            pltpu.VMEM((MS,), jnp.float32),
            pltpu.VMEM((MS,), jnp.float32),
            pltpu.VMEM((MS,), jnp.float32),
            pltpu.VMEM((MS,), jnp.int32),
            pltpu.VMEM((MS,), jnp.float32),
            pltpu.VMEM((RPW, 128), jnp.float32),
            pltpu.VMEM((NS,), jnp.float32),
            pltpu.VMEM((NS * NS,), jnp.float32),
            pltpu.VMEM((NS,), jnp.float32),
            pltpu.VMEM_SHARED((M,), jnp.float32),
            pltpu.VMEM_SHARED((M,), jnp.float32),
            pltpu.VMEM_SHARED((M,), jnp.float32),
            pltpu.VMEM_SHARED((NS * NS,), jnp.float32),
        ],
    )
    def k(err_hbm, rid_hbm, el_hbm, es_hbm, c0_hbm, pms_hbm, out_hbm,
          rid_v, err_v, ones_v, sums_t, cnts_t, el_t, es_t, c0_t,
          lpr_t, lp_v, ss_t, pall_t, pms_v, sums_sh, cnts_sh, lpr_sh,
          ss_sh):
        cid = lax.axis_index("c")
        sid = lax.axis_index("s")

        @pl.when(cid == 0)
        def _work():
            for j in range(128 // NS):
                ones_v[pl.ds(j * NS, NS)] = jnp.full((NS,), 1.0, jnp.float32)
            for j in range(MS // NS):
                sums_t[pl.ds(j * NS, NS)] = jnp.zeros((NS,), jnp.float32)
            base_m = sid * MS
            pltpu.sync_copy(sums_t, sums_sh.at[pl.ds(base_m, MS)])
            pltpu.sync_copy(sums_t, cnts_sh.at[pl.ds(base_m, MS)])

            row0 = sid * RPW
            pltpu.sync_copy(rid_hbm.at[pl.ds(row0, RPW)], rid_v)
            pltpu.sync_copy(err_hbm.at[pl.ds(row0, RPW)], err_v)
            pms_v[...] = jnp.zeros((NS,), jnp.float32)
            pltpu.sync_copy(pms_hbm, pms_v.at[pl.ds(0, 1)])
            plsc.subcore_barrier()

            for r in range(RPW):
                pltpu.sync_copy(err_v.at[r], sums_sh.at[rid_v.at[r]],
                                add=True)
                pltpu.sync_copy(ones_v, cnts_sh.at[rid_v.at[r]], add=True)
            plsc.subcore_barrier()

            pltpu.sync_copy(sums_sh.at[pl.ds(base_m, MS)], sums_t)
            pltpu.sync_copy(cnts_sh.at[pl.ds(base_m, MS)], cnts_t)
            pltpu.sync_copy(el_hbm.at[pl.ds(base_m, MS)], el_t)
            pltpu.sync_copy(es_hbm.at[pl.ds(base_m, MS)], es_t)
            pltpu.sync_copy(c0_hbm.at[pl.ds(base_m, MS)], c0_t)
            for j in range(MS // NS):
                sl = pl.ds(j * NS, NS)
                s = sums_t[sl]
                c = cnts_t[sl]
                el = el_t[sl]
                es = es_t[sl]
                c0 = c0_t[sl].astype(jnp.float32)
                means = s / jnp.maximum(c, 1.0)
                pres = jnp.minimum(c, 1.0)
                old = jnp.minimum(c0, 1.0)
                ema_l = BETA_LONG * el + (1.0 - BETA_LONG) * means
                ema_s = BETA_SHORT * es + (1.0 - BETA_SHORT) * means
                upd_l = (1.0 - old) * means + old * ema_l
                upd_s = (1.0 - old) * means + old * ema_s
                nl = (1.0 - pres) * el + pres * upd_l
                nsh = (1.0 - pres) * es + pres * upd_s
                lpr_t[sl] = (pres * old) * jnp.maximum(nl - nsh, 0.0)
            pltpu.sync_copy(lpr_t, lpr_sh.at[pl.ds(base_m, MS)])
            plsc.subcore_barrier()

            for r in range(RPW):
                pltpu.sync_copy(lpr_sh.at[rid_v.at[r]], lp_v.at[r])

            acc = jnp.zeros((NS,), jnp.float32)
            for r in range(RPW):
                for j in range(128 // NS):
                    v = lp_v[r, pl.ds(j * NS, NS)]
                    acc = acc + v * v
            ss_t[...] = acc
            pltpu.sync_copy(ss_t, ss_sh.at[pl.ds(sid * NS, NS)])
            plsc.subcore_barrier()

            pltpu.sync_copy(ss_sh, pall_t)
            tot = jnp.zeros((NS,), jnp.float32)
            for j in range(NS):
                tot = tot + pall_t[pl.ds(j * NS, NS)]
            sumsq = lax.reduce_sum_p.bind(tot, axes=(0,))
            pmsval = lax.reduce_sum_p.bind(pms_v[...], axes=(0,))
            ms = 0.99 * pmsval + (0.01 / B) * sumsq
            x = jnp.full((NS,), ms + 1e-8, jnp.float32)
            xi = lax.bitcast_convert_type(x, jnp.int32)
            seed = jnp.full((NS,), 0x5F3759DF, jnp.int32) - (xi >> 1)
            r0 = lax.bitcast_convert_type(seed, jnp.float32)
            for _ in range(3):
                r0 = r0 * (1.5 - 0.5 * x * r0 * r0)
            scale = ALPHA_LP / (x * r0 + 1e-8)
            for r in range(RPW):
                for j in range(128 // NS):
                    sl = pl.ds(j * NS, NS)
                    lp_v[r, sl] = lp_v[r, sl] * scale
            pltpu.sync_copy(lp_v, out_hbm.at[pl.ds(row0, RPW)])

    return k(err2d, rids2d, ema_long, ema_short, counts, prev_ms)


def kernel(obs, next_obs, actions, rids, ema_long, ema_short, counts,
           W_enc, b_enc, W_fwd, b_fwd, prev_ms):
    err = _err_tc(obs, next_obs, actions, W_enc, b_enc, W_fwd, b_fwd)
    out2d = _sc_middle(err.reshape(ROWS, 128), rids.reshape(ROWS, 128),
                       ema_long, ema_short, counts, prev_ms)
    return out2d.reshape(B)


# speedup vs baseline: 1.1132x; 1.1132x over previous
import dataclasses
import functools

import jax
import jax.numpy as jnp
from jax import lax
from jax.experimental import pallas as pl
from jax.experimental.pallas import tpu as pltpu
from jax.experimental.pallas import tpu_sc as plsc

B, D, P, A, M = 16384, 512, 256, 32, 4096
BETA_LONG, BETA_SHORT, ALPHA_LP = 0.995, 0.9, 0.5

ERR_TILE = 2048


def _err_body(obs_ref, nobs_ref, act_ref, we_ref, be_ref, wf1_ref, wf2_ref,
              bf_ref, err_ref):
    obs = obs_ref[...]
    nobs = nobs_ref[...]
    phi_t = jnp.maximum(obs @ we_ref[...] + be_ref[...], 0.0)
    phi_tp1 = jnp.maximum(nobs @ we_ref[...] + be_ref[...], 0.0)
    pred = phi_t @ wf1_ref[...] + act_ref[...] @ wf2_ref[...] + bf_ref[...]
    d = pred - phi_tp1
    err_ref[...] = jnp.sum(d * d, axis=1, keepdims=True) * (1.0 / P)


def _err_tc(obs, next_obs, actions, W_enc, b_enc, W_fwd, b_fwd):
    grid = B // ERR_TILE
    return pl.pallas_call(
        _err_body,
        grid=(grid,),
        in_specs=[
            pl.BlockSpec((ERR_TILE, D), lambda i: (i, 0)),
            pl.BlockSpec((ERR_TILE, D), lambda i: (i, 0)),
            pl.BlockSpec((ERR_TILE, A), lambda i: (i, 0)),
            pl.BlockSpec((D, P), lambda i: (0, 0)),
            pl.BlockSpec((1, P), lambda i: (0, 0)),
            pl.BlockSpec((P, P), lambda i: (0, 0)),
            pl.BlockSpec((A, P), lambda i: (0, 0)),
            pl.BlockSpec((1, P), lambda i: (0, 0)),
        ],
        out_specs=pl.BlockSpec((ERR_TILE, 1), lambda i: (i, 0)),
        out_shape=jax.ShapeDtypeStruct((B, 1), jnp.float32),
    )(obs, next_obs, actions, W_enc, b_enc.reshape(1, P), W_fwd[:P],
      W_fwd[P:], b_fwd.reshape(1, P))


NS = 16
ROWS = B // 128
RPW = ROWS // NS
MS = M // NS


def _sc_middle(err2d, rids2d, ema_long, ema_short, counts, prev_ms):
    mesh = plsc.VectorSubcoreMesh(core_axis_name="c", subcore_axis_name="s")
    cp = pltpu.CompilerParams()
    if "needs_layout_passes" in pltpu.CompilerParams.__dataclass_fields__:
        cp = dataclasses.replace(cp, needs_layout_passes=False)

    @functools.partial(
        pl.kernel, mesh=mesh, compiler_params=cp,
        out_type=jax.ShapeDtypeStruct((ROWS, 128), jnp.float32),
        scratch_types=[
            pltpu.VMEM((RPW, 128), jnp.int32),
            pltpu.VMEM((RPW, 128), jnp.float32),
            pltpu.VMEM((128,), jnp.float32),
            pltpu.VMEM((MS,), jnp.float32),
            pltpu.VMEM((MS,), jnp.float32),
            pltpu.VMEM((MS,), jnp.float32),
            pltpu.VMEM((MS,), jnp.float32),
            pltpu.VMEM((MS,), jnp.int32),
            pltpu.VMEM((MS,), jnp.float32),
            pltpu.VMEM((RPW, 128), jnp.float32),
            pltpu.VMEM((NS,), jnp.float32),
            pltpu.VMEM((NS * NS,), jnp.float32),
            pltpu.VMEM((NS,), jnp.float32),
            pltpu.VMEM_SHARED((M,), jnp.float32),
            pltpu.VMEM_SHARED((M,), jnp.float32),
            pltpu.VMEM_SHARED((M,), jnp.float32),
            pltpu.VMEM_SHARED((NS * NS,), jnp.float32),
        ],
    )
    def k(err_hbm, rid_hbm, el_hbm, es_hbm, c0_hbm, pms_hbm, out_hbm,
          rid_v, err_v, ones_v, sums_t, cnts_t, el_t, es_t, c0_t,
          lpr_t, lp_v, ss_t, pall_t, pms_v, sums_sh, cnts_sh, lpr_sh,
          ss_sh):
        cid = lax.axis_index("c")
        sid = lax.axis_index("s")

        @pl.when(cid == 0)
        def _work():
            for j in range(128 // NS):
                ones_v[pl.ds(j * NS, NS)] = jnp.full((NS,), 1.0, jnp.float32)
            for j in range(MS // NS):
                sums_t[pl.ds(j * NS, NS)] = jnp.zeros((NS,), jnp.float32)
            base_m = sid * MS
            pltpu.sync_copy(sums_t, sums_sh.at[pl.ds(base_m, MS)])
            pltpu.sync_copy(sums_t, cnts_sh.at[pl.ds(base_m, MS)])

            row0 = sid * RPW
            pltpu.sync_copy(rid_hbm.at[pl.ds(row0, RPW)], rid_v)
            pltpu.sync_copy(err_hbm.at[pl.ds(row0, RPW)], err_v)
            pms_v[...] = jnp.zeros((NS,), jnp.float32)
            pltpu.sync_copy(pms_hbm, pms_v.at[pl.ds(0, 1)])
            plsc.subcore_barrier()

            for r in range(RPW):
                pltpu.sync_copy(err_v.at[r], sums_sh.at[rid_v.at[r]],
                                add=True)
                pltpu.sync_copy(ones_v, cnts_sh.at[rid_v.at[r]], add=True)
            plsc.subcore_barrier()

            pltpu.sync_copy(sums_sh.at[pl.ds(base_m, MS)], sums_t)
            pltpu.sync_copy(cnts_sh.at[pl.ds(base_m, MS)], cnts_t)
            pltpu.sync_copy(el_hbm.at[pl.ds(base_m, MS)], el_t)
            pltpu.sync_copy(es_hbm.at[pl.ds(base_m, MS)], es_t)
            pltpu.sync_copy(c0_hbm.at[pl.ds(base_m, MS)], c0_t)
            for j in range(MS // NS):
                sl = pl.ds(j * NS, NS)
                s = sums_t[sl]
                c = cnts_t[sl]
                el = el_t[sl]
                es = es_t[sl]
                c0 = c0_t[sl].astype(jnp.float32)
                means = s / jnp.maximum(c, 1.0)
                pres = jnp.minimum(c, 1.0)
                old = jnp.minimum(c0, 1.0)
                ema_l = BETA_LONG * el + (1.0 - BETA_LONG) * means
                ema_s = BETA_SHORT * es + (1.0 - BETA_SHORT) * means
                upd_l = (1.0 - old) * means + old * ema_l
                upd_s = (1.0 - old) * means + old * ema_s
                nl = (1.0 - pres) * el + pres * upd_l
                nsh = (1.0 - pres) * es + pres * upd_s
                lpr_t[sl] = (pres * old) * jnp.maximum(nl - nsh, 0.0)
            pltpu.sync_copy(lpr_t, lpr_sh.at[pl.ds(base_m, MS)])
            plsc.subcore_barrier()

            for r in range(RPW):
                pltpu.sync_copy(lpr_sh.at[rid_v.at[r]], lp_v.at[r])

            acc = jnp.zeros((NS,), jnp.float32)
            for r in range(RPW):
                for j in range(128 // NS):
                    v = lp_v[r, pl.ds(j * NS, NS)]
                    acc = acc + v * v
            ss_t[...] = acc
            pltpu.sync_copy(ss_t, ss_sh.at[pl.ds(sid * NS, NS)])
            plsc.subcore_barrier()

            pltpu.sync_copy(ss_sh, pall_t)
            tot = jnp.zeros((NS,), jnp.float32)
            for j in range(NS):
                tot = tot + pall_t[pl.ds(j * NS, NS)]
            sumsq = lax.reduce_sum_p.bind(tot, axes=(0,))
            pmsval = lax.reduce_sum_p.bind(pms_v[...], axes=(0,))
            ms = 0.99 * pmsval + (0.01 / B) * sumsq
            x = jnp.full((NS,), ms + 1e-8, jnp.float32)
            xi = lax.bitcast_convert_type(x, jnp.int32)
            seed = jnp.full((NS,), 0x5F3759DF, jnp.int32) - (xi >> 1)
            r0 = lax.bitcast_convert_type(seed, jnp.float32)
            for _ in range(3):
                r0 = r0 * (1.5 - 0.5 * x * r0 * r0)
            scale = ALPHA_LP / (x * r0 + 1e-8)
            for r in range(RPW):
                for j in range(128 // NS):
                    sl = pl.ds(j * NS, NS)
                    lp_v[r, sl] = lp_v[r, sl] * scale
            pltpu.sync_copy(lp_v, out_hbm.at[pl.ds(row0, RPW)])

    return k(err2d, rids2d, ema_long, ema_short, counts, prev_ms)


def kernel(obs, next_obs, actions, rids, ema_long, ema_short, counts,
           W_enc, b_enc, W_fwd, b_fwd, prev_ms):
    err = _err_tc(obs, next_obs, actions, W_enc, b_enc, W_fwd, b_fwd)
    out2d = _sc_middle(err.reshape(ROWS, 128), rids.reshape(ROWS, 128),
                       ema_long, ema_short, counts, prev_ms)
    return out2d.reshape(B)


# speedup vs baseline: 1.1159x; 1.0024x over previous
import dataclasses
import functools

import jax
import jax.numpy as jnp
from jax import lax
from jax.experimental import pallas as pl
from jax.experimental.pallas import tpu as pltpu
from jax.experimental.pallas import tpu_sc as plsc

B, D, P, A, M = 16384, 512, 256, 32, 4096
BETA_LONG, BETA_SHORT, ALPHA_LP = 0.995, 0.9, 0.5

ERR_TILE = 2048


def _err_body(obs_ref, nobs_ref, act_ref, we_ref, be_ref, wf1_ref, wf2_ref,
              bf_ref, err_ref):
    obs = obs_ref[...]
    nobs = nobs_ref[...]
    phi_t = jnp.maximum(obs @ we_ref[...] + be_ref[...], 0.0)
    phi_tp1 = jnp.maximum(nobs @ we_ref[...] + be_ref[...], 0.0)
    pred = phi_t @ wf1_ref[...] + act_ref[...] @ wf2_ref[...] + bf_ref[...]
    d = pred - phi_tp1
    err_ref[...] = jnp.sum(d * d, axis=1, keepdims=True) * (1.0 / P)


def _err_tc(obs, next_obs, actions, W_enc, b_enc, W_fwd, b_fwd):
    grid = B // ERR_TILE
    return pl.pallas_call(
        _err_body,
        grid=(grid,),
        in_specs=[
            pl.BlockSpec((ERR_TILE, D), lambda i: (i, 0)),
            pl.BlockSpec((ERR_TILE, D), lambda i: (i, 0)),
            pl.BlockSpec((ERR_TILE, A), lambda i: (i, 0)),
            pl.BlockSpec((D, P), lambda i: (0, 0)),
            pl.BlockSpec((1, P), lambda i: (0, 0)),
            pl.BlockSpec((P, P), lambda i: (0, 0)),
            pl.BlockSpec((A, P), lambda i: (8, 0)),
            pl.BlockSpec((1, P), lambda i: (0, 0)),
        ],
        out_specs=pl.BlockSpec((ERR_TILE, 1), lambda i: (i, 0)),
        out_shape=jax.ShapeDtypeStruct((B, 1), jnp.float32),
    )(obs, next_obs, actions, W_enc, b_enc.reshape(1, P), W_fwd, W_fwd,
      b_fwd.reshape(1, P))


NS = 16
ROWS = B // 128
RPW = ROWS // NS
MS = M // NS


def _sc_middle(err2d, rids2d, ema_long, ema_short, counts, prev_ms):
    mesh = plsc.VectorSubcoreMesh(core_axis_name="c", subcore_axis_name="s")
    cp = pltpu.CompilerParams()
    if "needs_layout_passes" in pltpu.CompilerParams.__dataclass_fields__:
        cp = dataclasses.replace(cp, needs_layout_passes=False)

    @functools.partial(
        pl.kernel, mesh=mesh, compiler_params=cp,
        out_type=jax.ShapeDtypeStruct((ROWS, 128), jnp.float32),
        scratch_types=[
            pltpu.VMEM((RPW, 128), jnp.int32),
            pltpu.VMEM((RPW, 128), jnp.float32),
            pltpu.VMEM((128,), jnp.float32),
            pltpu.VMEM((MS,), jnp.float32),
            pltpu.VMEM((MS,), jnp.float32),
            pltpu.VMEM((MS,), jnp.float32),
            pltpu.VMEM((MS,), jnp.float32),
            pltpu.VMEM((MS,), jnp.int32),
            pltpu.VMEM((MS,), jnp.float32),
            pltpu.VMEM((RPW, 128), jnp.float32),
            pltpu.VMEM((NS,), jnp.float32),
            pltpu.VMEM((NS * NS,), jnp.float32),
            pltpu.VMEM((NS,), jnp.float32),
            pltpu.VMEM_SHARED((M,), jnp.float32),
            pltpu.VMEM_SHARED((M,), jnp.float32),
            pltpu.VMEM_SHARED((M,), jnp.float32),
            pltpu.VMEM_SHARED((NS * NS,), jnp.float32),
        ],
    )
    def k(err_hbm, rid_hbm, el_hbm, es_hbm, c0_hbm, pms_hbm, out_hbm,
          rid_v, err_v, ones_v, sums_t, cnts_t, el_t, es_t, c0_t,
          lpr_t, lp_v, ss_t, pall_t, pms_v, sums_sh, cnts_sh, lpr_sh,
          ss_sh):
        cid = lax.axis_index("c")
        sid = lax.axis_index("s")

        @pl.when(cid == 0)
        def _work():
            for j in range(128 // NS):
                ones_v[pl.ds(j * NS, NS)] = jnp.full((NS,), 1.0, jnp.float32)
            for j in range(MS // NS):
                sums_t[pl.ds(j * NS, NS)] = jnp.zeros((NS,), jnp.float32)
            base_m = sid * MS
            pltpu.sync_copy(sums_t, sums_sh.at[pl.ds(base_m, MS)])
            pltpu.sync_copy(sums_t, cnts_sh.at[pl.ds(base_m, MS)])

            row0 = sid * RPW
            for r in range(RPW):
                pltpu.sync_copy(rid_hbm.at[pl.ds((row0 + r) * 128, 128)],
                                rid_v.at[r])
            pltpu.sync_copy(err_hbm.at[pl.ds(row0, RPW)], err_v)
            pms_v[...] = jnp.zeros((NS,), jnp.float32)
            pltpu.sync_copy(pms_hbm, pms_v.at[pl.ds(0, 1)])
            plsc.subcore_barrier()

            for r in range(RPW):
                pltpu.sync_copy(err_v.at[r], sums_sh.at[rid_v.at[r]],
                                add=True)
                pltpu.sync_copy(ones_v, cnts_sh.at[rid_v.at[r]], add=True)
            plsc.subcore_barrier()

            pltpu.sync_copy(sums_sh.at[pl.ds(base_m, MS)], sums_t)
            pltpu.sync_copy(cnts_sh.at[pl.ds(base_m, MS)], cnts_t)
            pltpu.sync_copy(el_hbm.at[pl.ds(base_m, MS)], el_t)
            pltpu.sync_copy(es_hbm.at[pl.ds(base_m, MS)], es_t)
            pltpu.sync_copy(c0_hbm.at[pl.ds(base_m, MS)], c0_t)
            for j in range(MS // NS):
                sl = pl.ds(j * NS, NS)
                s = sums_t[sl]
                c = cnts_t[sl]
                el = el_t[sl]
                es = es_t[sl]
                c0 = c0_t[sl].astype(jnp.float32)
                means = s / jnp.maximum(c, 1.0)
                pres = jnp.minimum(c, 1.0)
                old = jnp.minimum(c0, 1.0)
                ema_l = BETA_LONG * el + (1.0 - BETA_LONG) * means
                ema_s = BETA_SHORT * es + (1.0 - BETA_SHORT) * means
                upd_l = (1.0 - old) * means + old * ema_l
                upd_s = (1.0 - old) * means + old * ema_s
                nl = (1.0 - pres) * el + pres * upd_l
                nsh = (1.0 - pres) * es + pres * upd_s
                lpr_t[sl] = (pres * old) * jnp.maximum(nl - nsh, 0.0)
            pltpu.sync_copy(lpr_t, lpr_sh.at[pl.ds(base_m, MS)])
            plsc.subcore_barrier()

            for r in range(RPW):
                pltpu.sync_copy(lpr_sh.at[rid_v.at[r]], lp_v.at[r])

            acc = jnp.zeros((NS,), jnp.float32)
            for r in range(RPW):
                for j in range(128 // NS):
                    v = lp_v[r, pl.ds(j * NS, NS)]
                    acc = acc + v * v
            ss_t[...] = acc
            pltpu.sync_copy(ss_t, ss_sh.at[pl.ds(sid * NS, NS)])
            plsc.subcore_barrier()

            pltpu.sync_copy(ss_sh, pall_t)
            tot = jnp.zeros((NS,), jnp.float32)
            for j in range(NS):
                tot = tot + pall_t[pl.ds(j * NS, NS)]
            sumsq = lax.reduce_sum_p.bind(tot, axes=(0,))
            pmsval = lax.reduce_sum_p.bind(pms_v[...], axes=(0,))
            ms = 0.99 * pmsval + (0.01 / B) * sumsq
            x = jnp.full((NS,), ms + 1e-8, jnp.float32)
            xi = lax.bitcast_convert_type(x, jnp.int32)
            seed = jnp.full((NS,), 0x5F3759DF, jnp.int32) - (xi >> 1)
            r0 = lax.bitcast_convert_type(seed, jnp.float32)
            for _ in range(3):
                r0 = r0 * (1.5 - 0.5 * x * r0 * r0)
            scale = ALPHA_LP / (x * r0 + 1e-8)
            for r in range(RPW):
                for j in range(128 // NS):
                    sl = pl.ds(j * NS, NS)
                    lp_v[r, sl] = lp_v[r, sl] * scale
            pltpu.sync_copy(lp_v, out_hbm.at[pl.ds(row0, RPW)])

    return k(err2d, rids2d, ema_long, ema_short, counts, prev_ms)


def kernel(obs, next_obs, actions, rids, ema_long, ema_short, counts,
           W_enc, b_enc, W_fwd, b_fwd, prev_ms):
    err = _err_tc(obs, next_obs, actions, W_enc, b_enc, W_fwd, b_fwd)
    out2d = _sc_middle(err.reshape(ROWS, 128), rids,
                       ema_long, ema_short, counts, prev_ms)
    return out2d.reshape(B)


# speedup vs baseline: 1.4786x; 1.3250x over previous
import dataclasses
import functools

import jax
import jax.numpy as jnp
from jax import lax
from jax.experimental import pallas as pl
from jax.experimental.pallas import tpu as pltpu
from jax.experimental.pallas import tpu_sc as plsc

B, D, P, A, M = 16384, 512, 256, 32, 4096
BETA_LONG, BETA_SHORT, ALPHA_LP = 0.995, 0.9, 0.5

ERR_TILE = 2048


def _err_body(obs_ref, nobs_ref, actT_ref, we_ref, be_ref, wf1_ref, wf2_ref,
              bf_ref, err_ref):
    obs = obs_ref[...]
    nobs = nobs_ref[...]
    phi_t = jnp.maximum(obs @ we_ref[...] + be_ref[...], 0.0)
    phi_tp1 = jnp.maximum(nobs @ we_ref[...] + be_ref[...], 0.0)
    act_term = lax.dot_general(actT_ref[...], wf2_ref[...],
                               (((0,), (0,)), ((), ())),
                               preferred_element_type=jnp.float32)
    pred = phi_t @ wf1_ref[...] + act_term + bf_ref[...]
    d = pred - phi_tp1
    errcol = jnp.sum(d * d, axis=1, keepdims=True) * (1.0 / P)
    bidx = lax.broadcasted_iota(jnp.int32, (ERR_TILE, 128), 0)
    lidx = lax.broadcasted_iota(jnp.int32, (ERR_TILE, 128), 1)
    pick = (bidx - (bidx // 128) * 128 == lidx).astype(jnp.float32)
    sidx = lax.broadcasted_iota(jnp.int32, (ERR_TILE // 128, ERR_TILE), 0)
    b2 = lax.broadcasted_iota(jnp.int32, (ERR_TILE // 128, ERR_TILE), 1)
    grp = (b2 // 128 == sidx).astype(jnp.float32)
    err_ref[...] = grp @ (errcol * pick)


def _err_tc(obs, next_obs, actions, W_enc, b_enc, W_fwd, b_fwd):
    grid = B // ERR_TILE
    return pl.pallas_call(
        _err_body,
        grid=(grid,),
        in_specs=[
            pl.BlockSpec((ERR_TILE, D), lambda i: (i, 0)),
            pl.BlockSpec((ERR_TILE, D), lambda i: (i, 0)),
            pl.BlockSpec((A, ERR_TILE), lambda i: (0, i)),
            pl.BlockSpec((D, P), lambda i: (0, 0)),
            pl.BlockSpec((1, P), lambda i: (0, 0)),
            pl.BlockSpec((P, P), lambda i: (0, 0)),
            pl.BlockSpec((A, P), lambda i: (8, 0)),
            pl.BlockSpec((1, P), lambda i: (0, 0)),
        ],
        out_specs=pl.BlockSpec((ERR_TILE // 128, 128), lambda i: (i, 0)),
        out_shape=jax.ShapeDtypeStruct((ROWS, 128), jnp.float32),
    )(obs, next_obs, actions.T, W_enc, b_enc.reshape(1, P), W_fwd, W_fwd,
      b_fwd.reshape(1, P))


NS = 16
ROWS = B // 128
RPW = ROWS // NS
MS = M // NS


def _sc_middle(err2d, rids2d, ema_long, ema_short, counts, prev_ms):
    mesh = plsc.VectorSubcoreMesh(core_axis_name="c", subcore_axis_name="s")
    cp = pltpu.CompilerParams()
    if "needs_layout_passes" in pltpu.CompilerParams.__dataclass_fields__:
        cp = dataclasses.replace(cp, needs_layout_passes=False)

    @functools.partial(
        pl.kernel, mesh=mesh, compiler_params=cp,
        out_type=jax.ShapeDtypeStruct((ROWS, 128), jnp.float32),
        scratch_types=[
            pltpu.VMEM((RPW, 128), jnp.int32),
            pltpu.VMEM((RPW, 128), jnp.float32),
            pltpu.VMEM((128,), jnp.float32),
            pltpu.VMEM((MS,), jnp.float32),
            pltpu.VMEM((MS,), jnp.float32),
            pltpu.VMEM((MS,), jnp.float32),
            pltpu.VMEM((MS,), jnp.float32),
            pltpu.VMEM((MS,), jnp.int32),
            pltpu.VMEM((MS,), jnp.float32),
            pltpu.VMEM((RPW, 128), jnp.float32),
            pltpu.VMEM((NS,), jnp.float32),
            pltpu.VMEM((NS * NS,), jnp.float32),
            pltpu.VMEM((NS,), jnp.float32),
            pltpu.VMEM_SHARED((M,), jnp.float32),
            pltpu.VMEM_SHARED((M,), jnp.float32),
            pltpu.VMEM_SHARED((M,), jnp.float32),
            pltpu.VMEM_SHARED((NS * NS,), jnp.float32),
        ],
    )
    def k(err_hbm, rid_hbm, el_hbm, es_hbm, c0_hbm, pms_hbm, out_hbm,
          rid_v, err_v, ones_v, sums_t, cnts_t, el_t, es_t, c0_t,
          lpr_t, lp_v, ss_t, pall_t, pms_v, sums_sh, cnts_sh, lpr_sh,
          ss_sh):
        cid = lax.axis_index("c")
        sid = lax.axis_index("s")

        @pl.when(cid == 0)
        def _work():
            for j in range(128 // NS):
                ones_v[pl.ds(j * NS, NS)] = jnp.full((NS,), 1.0, jnp.float32)
            for j in range(MS // NS):
                sums_t[pl.ds(j * NS, NS)] = jnp.zeros((NS,), jnp.float32)
            base_m = sid * MS
            pltpu.sync_copy(sums_t, sums_sh.at[pl.ds(base_m, MS)])
            pltpu.sync_copy(sums_t, cnts_sh.at[pl.ds(base_m, MS)])

            row0 = sid * RPW
            pltpu.sync_copy(rid_hbm.at[pl.ds(row0, RPW)], rid_v)
            pltpu.sync_copy(err_hbm.at[pl.ds(row0, RPW)], err_v)
            pms_v[...] = jnp.zeros((NS,), jnp.float32)
            pltpu.sync_copy(pms_hbm, pms_v.at[pl.ds(0, 1)])
            plsc.subcore_barrier()

            for r in range(RPW):
                pltpu.sync_copy(err_v.at[r], sums_sh.at[rid_v.at[r]],
                                add=True)
                pltpu.sync_copy(ones_v, cnts_sh.at[rid_v.at[r]], add=True)
            plsc.subcore_barrier()

            pltpu.sync_copy(sums_sh.at[pl.ds(base_m, MS)], sums_t)
            pltpu.sync_copy(cnts_sh.at[pl.ds(base_m, MS)], cnts_t)
            pltpu.sync_copy(el_hbm.at[pl.ds(base_m, MS)], el_t)
            pltpu.sync_copy(es_hbm.at[pl.ds(base_m, MS)], es_t)
            pltpu.sync_copy(c0_hbm.at[pl.ds(base_m, MS)], c0_t)
            for j in range(MS // NS):
                sl = pl.ds(j * NS, NS)
                s = sums_t[sl]
                c = cnts_t[sl]
                el = el_t[sl]
                es = es_t[sl]
                c0 = c0_t[sl].astype(jnp.float32)
                means = s / jnp.maximum(c, 1.0)
                pres = jnp.minimum(c, 1.0)
                old = jnp.minimum(c0, 1.0)
                ema_l = BETA_LONG * el + (1.0 - BETA_LONG) * means
                ema_s = BETA_SHORT * es + (1.0 - BETA_SHORT) * means
                upd_l = (1.0 - old) * means + old * ema_l
                upd_s = (1.0 - old) * means + old * ema_s
                nl = (1.0 - pres) * el + pres * upd_l
                nsh = (1.0 - pres) * es + pres * upd_s
                lpr_t[sl] = (pres * old) * jnp.maximum(nl - nsh, 0.0)
            pltpu.sync_copy(lpr_t, lpr_sh.at[pl.ds(base_m, MS)])
            plsc.subcore_barrier()

            for r in range(RPW):
                pltpu.sync_copy(lpr_sh.at[rid_v.at[r]], lp_v.at[r])

            acc = jnp.zeros((NS,), jnp.float32)
            for r in range(RPW):
                for j in range(128 // NS):
                    v = lp_v[r, pl.ds(j * NS, NS)]
                    acc = acc + v * v
            ss_t[...] = acc
            pltpu.sync_copy(ss_t, ss_sh.at[pl.ds(sid * NS, NS)])
            plsc.subcore_barrier()

            pltpu.sync_copy(ss_sh, pall_t)
            tot = jnp.zeros((NS,), jnp.float32)
            for j in range(NS):
                tot = tot + pall_t[pl.ds(j * NS, NS)]
            sumsq = lax.reduce_sum_p.bind(tot, axes=(0,))
            pmsval = lax.reduce_sum_p.bind(pms_v[...], axes=(0,))
            ms = 0.99 * pmsval + (0.01 / B) * sumsq
            x = jnp.full((NS,), ms + 1e-8, jnp.float32)
            xi = lax.bitcast_convert_type(x, jnp.int32)
            seed = jnp.full((NS,), 0x5F3759DF, jnp.int32) - (xi >> 1)
            r0 = lax.bitcast_convert_type(seed, jnp.float32)
            for _ in range(3):
                r0 = r0 * (1.5 - 0.5 * x * r0 * r0)
            scale = ALPHA_LP / (x * r0 + 1e-8)
            for r in range(RPW):
                for j in range(128 // NS):
                    sl = pl.ds(j * NS, NS)
                    lp_v[r, sl] = lp_v[r, sl] * scale
            pltpu.sync_copy(lp_v, out_hbm.at[pl.ds(row0, RPW)])

    return k(err2d, rids2d, ema_long, ema_short, counts, prev_ms)


def kernel(obs, next_obs, actions, rids, ema_long, ema_short, counts,
           W_enc, b_enc, W_fwd, b_fwd, prev_ms):
    err2d = _err_tc(obs, next_obs, actions, W_enc, b_enc, W_fwd, b_fwd)
    out2d = _sc_middle(err2d, rids.reshape(ROWS, 128),
                       ema_long, ema_short, counts, prev_ms)
    return out2d.reshape(B)


# speedup vs baseline: 1.6125x; 1.0905x over previous
import dataclasses
import functools

import jax
import jax.numpy as jnp
from jax import lax
from jax.experimental import pallas as pl
from jax.experimental.pallas import tpu as pltpu
from jax.experimental.pallas import tpu_sc as plsc

B, D, P, A, M = 16384, 512, 256, 32, 4096
BETA_LONG, BETA_SHORT, ALPHA_LP = 0.995, 0.9, 0.5

ERR_TILE = 2048


def _err_body(obs_ref, nobs_ref, actT_ref, we_ref, be_ref, wf1_ref, wf2_ref,
              bf_ref, err_ref):
    obs = obs_ref[...]
    nobs = nobs_ref[...]
    phi_t = jnp.maximum(obs @ we_ref[...] + be_ref[...], 0.0)
    phi_tp1 = jnp.maximum(nobs @ we_ref[...] + be_ref[...], 0.0)
    act_term = lax.dot_general(actT_ref[...], wf2_ref[...],
                               (((0,), (0,)), ((), ())),
                               preferred_element_type=jnp.float32)
    pred = phi_t @ wf1_ref[...] + act_term + bf_ref[...]
    d = pred - phi_tp1
    errcol = jnp.sum(d * d, axis=1, keepdims=True) * (1.0 / P)
    bidx = lax.broadcasted_iota(jnp.int32, (ERR_TILE, 128), 0)
    lidx = lax.broadcasted_iota(jnp.int32, (ERR_TILE, 128), 1)
    pick = (bidx - (bidx // 128) * 128 == lidx).astype(jnp.float32)
    sidx = lax.broadcasted_iota(jnp.int32, (ERR_TILE // 128, ERR_TILE), 0)
    b2 = lax.broadcasted_iota(jnp.int32, (ERR_TILE // 128, ERR_TILE), 1)
    grp = (b2 // 128 == sidx).astype(jnp.float32)
    err_ref[...] = grp @ (errcol * pick)


def _err_tc(obs, next_obs, actions, W_enc, b_enc, W_fwd, b_fwd):
    grid = B // ERR_TILE
    return pl.pallas_call(
        _err_body,
        grid=(grid,),
        in_specs=[
            pl.BlockSpec((ERR_TILE, D), lambda i: (i, 0)),
            pl.BlockSpec((ERR_TILE, D), lambda i: (i, 0)),
            pl.BlockSpec((A, ERR_TILE), lambda i: (0, i)),
            pl.BlockSpec((D, P), lambda i: (0, 0)),
            pl.BlockSpec((1, P), lambda i: (0, 0)),
            pl.BlockSpec((P, P), lambda i: (0, 0)),
            pl.BlockSpec((A, P), lambda i: (8, 0)),
            pl.BlockSpec((1, P), lambda i: (0, 0)),
        ],
        out_specs=pl.BlockSpec((ERR_TILE // 128, 128), lambda i: (i, 0)),
        out_shape=jax.ShapeDtypeStruct((ROWS, 128), jnp.float32),
    )(obs, next_obs, actions.T, W_enc, b_enc.reshape(1, P), W_fwd, W_fwd,
      b_fwd.reshape(1, P))


NS = 16
ROWS = B // 128
RPW = ROWS // NS
MS = M // NS


def _sc_middle(err2d, rids2d, ema_long, ema_short, counts, prev_ms):
    mesh = plsc.VectorSubcoreMesh(core_axis_name="c", subcore_axis_name="s")
    cp = pltpu.CompilerParams()
    if "needs_layout_passes" in pltpu.CompilerParams.__dataclass_fields__:
        cp = dataclasses.replace(cp, needs_layout_passes=False)

    @functools.partial(
        pl.kernel, mesh=mesh, compiler_params=cp,
        out_type=jax.ShapeDtypeStruct((ROWS, 128), jnp.float32),
        scratch_types=[
            pltpu.VMEM((RPW, 128), jnp.int32),
            pltpu.VMEM((RPW, 128), jnp.float32),
            pltpu.VMEM((128,), jnp.float32),
            pltpu.VMEM((MS,), jnp.float32),
            pltpu.VMEM((MS,), jnp.float32),
            pltpu.VMEM((MS,), jnp.float32),
            pltpu.VMEM((MS,), jnp.float32),
            pltpu.VMEM((MS,), jnp.int32),
            pltpu.VMEM((MS,), jnp.float32),
            pltpu.VMEM((RPW, 128), jnp.float32),
            pltpu.VMEM((NS,), jnp.float32),
            pltpu.VMEM((NS * NS,), jnp.float32),
            pltpu.VMEM((NS,), jnp.float32),
            pltpu.VMEM_SHARED((M,), jnp.float32),
            pltpu.VMEM_SHARED((M,), jnp.float32),
            pltpu.VMEM_SHARED((M,), jnp.float32),
            pltpu.VMEM_SHARED((NS * NS,), jnp.float32),
            pltpu.SemaphoreType.DMA,
            pltpu.SemaphoreType.DMA,
        ],
    )
    def k(err_hbm, rid_hbm, el_hbm, es_hbm, c0_hbm, pms_hbm, out_hbm,
          rid_v, err_v, ones_v, sums_t, cnts_t, el_t, es_t, c0_t,
          lpr_t, lp_v, ss_t, pall_t, pms_v, sums_sh, cnts_sh, lpr_sh,
          ss_sh, sem_in, sem_sc):
        cid = lax.axis_index("c")
        sid = lax.axis_index("s")

        @pl.when(cid == 0)
        def _work():
            base_m = sid * MS
            row0 = sid * RPW
            h_rid = pltpu.async_copy(rid_hbm.at[pl.ds(row0, RPW)], rid_v,
                                     sem_in)
            h_err = pltpu.async_copy(err_hbm.at[pl.ds(row0, RPW)], err_v,
                                     sem_in)
            h_el = pltpu.async_copy(el_hbm.at[pl.ds(base_m, MS)], el_t,
                                    sem_in)
            h_es = pltpu.async_copy(es_hbm.at[pl.ds(base_m, MS)], es_t,
                                    sem_in)
            h_c0 = pltpu.async_copy(c0_hbm.at[pl.ds(base_m, MS)], c0_t,
                                    sem_in)
            pms_v[pl.ds(0, NS)] = jnp.zeros((NS,), jnp.float32)
            h_pm = pltpu.async_copy(pms_hbm, pms_v.at[pl.ds(0, 1)], sem_in)
            for j in range(128 // NS):
                ones_v[pl.ds(j * NS, NS)] = jnp.full((NS,), 1.0, jnp.float32)
            for j in range(MS // NS):
                sums_t[pl.ds(j * NS, NS)] = jnp.zeros((NS,), jnp.float32)
            h_z1 = pltpu.async_copy(sums_t, sums_sh.at[pl.ds(base_m, MS)],
                                    sem_sc)
            h_z2 = pltpu.async_copy(sums_t, cnts_sh.at[pl.ds(base_m, MS)],
                                    sem_sc)
            h_z1.wait()
            h_z2.wait()
            h_rid.wait()
            h_err.wait()
            plsc.subcore_barrier()

            hs = []
            for r in range(RPW):
                hs.append(pltpu.async_copy(err_v.at[r],
                                           sums_sh.at[rid_v.at[r]],
                                           sem_sc, add=True))
                hs.append(pltpu.async_copy(ones_v,
                                           cnts_sh.at[rid_v.at[r]],
                                           sem_sc, add=True))
            for h in hs:
                h.wait()
            plsc.subcore_barrier()

            h_s = pltpu.async_copy(sums_sh.at[pl.ds(base_m, MS)], sums_t,
                                   sem_sc)
            h_c = pltpu.async_copy(cnts_sh.at[pl.ds(base_m, MS)], cnts_t,
                                   sem_sc)
            h_el.wait()
            h_es.wait()
            h_c0.wait()
            h_pm.wait()
            h_s.wait()
            h_c.wait()
            for j in range(MS // NS):
                sl = pl.ds(j * NS, NS)
                s = sums_t[sl]
                c = cnts_t[sl]
                el = el_t[sl]
                es = es_t[sl]
                c0 = c0_t[sl].astype(jnp.float32)
                means = s / jnp.maximum(c, 1.0)
                pres = jnp.minimum(c, 1.0)
                old = jnp.minimum(c0, 1.0)
                ema_l = BETA_LONG * el + (1.0 - BETA_LONG) * means
                ema_s = BETA_SHORT * es + (1.0 - BETA_SHORT) * means
                upd_l = (1.0 - old) * means + old * ema_l
                upd_s = (1.0 - old) * means + old * ema_s
                nl = (1.0 - pres) * el + pres * upd_l
                nsh = (1.0 - pres) * es + pres * upd_s
                lpr_t[sl] = (pres * old) * jnp.maximum(nl - nsh, 0.0)
            pltpu.sync_copy(lpr_t, lpr_sh.at[pl.ds(base_m, MS)])
            plsc.subcore_barrier()

            hg = [pltpu.async_copy(lpr_sh.at[rid_v.at[r]], lp_v.at[r],
                                   sem_sc) for r in range(RPW)]
            for h in hg:
                h.wait()

            acc = jnp.zeros((NS,), jnp.float32)
            for r in range(RPW):
                for j in range(128 // NS):
                    v = lp_v[r, pl.ds(j * NS, NS)]
                    acc = acc + v * v
            ss_t[...] = acc
            pltpu.sync_copy(ss_t, ss_sh.at[pl.ds(sid * NS, NS)])
            plsc.subcore_barrier()

            pltpu.sync_copy(ss_sh, pall_t)
            tot = jnp.zeros((NS,), jnp.float32)
            for j in range(NS):
                tot = tot + pall_t[pl.ds(j * NS, NS)]
            sumsq = lax.reduce_sum_p.bind(tot, axes=(0,))
            pmsval = lax.reduce_sum_p.bind(pms_v[...], axes=(0,))
            ms = 0.99 * pmsval + (0.01 / B) * sumsq
            x = jnp.full((NS,), ms + 1e-8, jnp.float32)
            xi = lax.bitcast_convert_type(x, jnp.int32)
            seed = jnp.full((NS,), 0x5F3759DF, jnp.int32) - (xi >> 1)
            r0 = lax.bitcast_convert_type(seed, jnp.float32)
            for _ in range(3):
                r0 = r0 * (1.5 - 0.5 * x * r0 * r0)
            scale = ALPHA_LP / (x * r0 + 1e-8)
            for r in range(RPW):
                for j in range(128 // NS):
                    sl = pl.ds(j * NS, NS)
                    lp_v[r, sl] = lp_v[r, sl] * scale
            pltpu.sync_copy(lp_v, out_hbm.at[pl.ds(row0, RPW)])

    return k(err2d, rids2d, ema_long, ema_short, counts, prev_ms)


def kernel(obs, next_obs, actions, rids, ema_long, ema_short, counts,
           W_enc, b_enc, W_fwd, b_fwd, prev_ms):
    err2d = _err_tc(obs, next_obs, actions, W_enc, b_enc, W_fwd, b_fwd)
    out2d = _sc_middle(err2d, rids.reshape(ROWS, 128),
                       ema_long, ema_short, counts, prev_ms)
    return out2d.reshape(B)
